# baseline (speedup 1.0000x reference)
import jax
import jax.numpy as jnp
from jax import lax
from jax.experimental import pallas as pl
import jax.experimental.pallas.tpu as pltpu
from jax.experimental.pallas import tpu_sc as plsc

_TOKENS = 2048
_D_IN = 768
_BASE = 128
_BT = 256
_SC_ROWS = 512
_NW = 32
_RPW = _SC_ROWS // _NW


def _tc_kernel(x_ref, w_ref, b_ref, o_ref):
    xs = jax.lax.dot_general(
        x_ref[:], w_ref[:],
        (((1,), (1,)), ((), ())),
        preferred_element_type=jnp.float32,
    ) + b_ref[:]
    x1 = xs[:, :_BASE]
    x2 = xs[:, _BASE:]
    for i in range(_BASE):
        o_ref[:, i * _BASE:(i + 1) * _BASE] = x1[:, i:i + 1] + x2


def _tc_expand(x, W, b2):
    return pl.pallas_call(
        _tc_kernel,
        grid=(_TOKENS // _BT,),
        in_specs=[
            pl.BlockSpec((_BT, _D_IN), lambda t: (t, 0)),
            pl.BlockSpec((2 * _BASE, _D_IN), lambda t: (0, 0)),
            pl.BlockSpec((1, 2 * _BASE), lambda t: (0, 0)),
        ],
        out_specs=pl.BlockSpec((_BT, _BASE * _BASE), lambda t: (t, 0)),
        out_shape=jax.ShapeDtypeStruct((_TOKENS, _BASE * _BASE), jnp.float32),
    )(x, W, b2)


def _xs_kernel(x_ref, w_ref, b_ref, o_ref):
    o_ref[:] = jax.lax.dot_general(
        x_ref[:], w_ref[:],
        (((1,), (1,)), ((), ())),
        preferred_element_type=jnp.float32,
    ) + b_ref[:]


def _compute_xs(x, W, b2):
    return pl.pallas_call(
        _xs_kernel,
        grid=(1,),
        in_specs=[
            pl.BlockSpec((_SC_ROWS, _D_IN), lambda i: (0, 0)),
            pl.BlockSpec((2 * _BASE, _D_IN), lambda i: (0, 0)),
            pl.BlockSpec((1, 2 * _BASE), lambda i: (0, 0)),
        ],
        out_specs=pl.BlockSpec((_SC_ROWS, 2 * _BASE), lambda i: (0, 0)),
        out_shape=jax.ShapeDtypeStruct((_SC_ROWS, 2 * _BASE), jnp.float32),
    )(x, W, b2)


def _sc_expand_body(xs_hbm, out_hbm, xs_v, buf, sem):
    wid = lax.axis_index("s") * 2 + lax.axis_index("c")
    base = wid * _RPW
    pltpu.sync_copy(xs_hbm.at[pl.ds(base, _RPW)], xs_v)

    def row_body(t, carry):
        slot = lax.rem(t, 2)

        @pl.when(t >= 2)
        def _wait_prev():
            pltpu.make_async_copy(
                buf.at[slot], out_hbm.at[base + t - 2], sem
            ).wait()

        x2vs = [xs_v[t, pl.ds(_BASE + jv * 16, 16)] for jv in range(8)]

        def col_body(iv, c2):
            x1v = xs_v[t, pl.ds(iv * 16, 16)]
            for l in range(16):
                s = x1v[l]
                off = (iv * 16 + l) * _BASE
                for jv in range(8):
                    buf[slot, pl.ds(off + jv * 16, 16)] = s + x2vs[jv]
            return c2

        lax.fori_loop(0, 8, col_body, 0)
        pltpu.make_async_copy(buf.at[slot], out_hbm.at[base + t], sem).start()
        return carry

    lax.fori_loop(0, _RPW, row_body, 0)
    for t in (_RPW - 2, _RPW - 1):
        pltpu.make_async_copy(
            buf.at[t % 2], out_hbm.at[base + t], sem
        ).wait()


def _sc_expand(xs):
    mesh = plsc.VectorSubcoreMesh(core_axis_name="c", subcore_axis_name="s")
    return pl.kernel(
        _sc_expand_body,
        out_type=jax.ShapeDtypeStruct((_SC_ROWS, _BASE * _BASE), jnp.float32),
        mesh=mesh,
        scratch_types=[
            pltpu.VMEM((_RPW, 2 * _BASE), jnp.float32),
            pltpu.VMEM((2, _BASE * _BASE), jnp.float32),
            pltpu.SemaphoreType.DMA,
        ],
        compiler_params=pltpu.CompilerParams(has_side_effects=True),
    )(xs)


def kernel(x, W, b):
    b2 = b.reshape(1, 2 * _BASE)
    xs = _compute_xs(x[:_SC_ROWS], W, b2)
    d = _sc_expand(xs)
    y = _tc_expand(x, W, b2)
    y, _ = jax.lax.optimization_barrier((y, d))
    return y


# speedup vs baseline: 1.0753x; 1.0753x over previous
import jax
import jax.numpy as jnp
from jax import lax
from jax.experimental import pallas as pl
import jax.experimental.pallas.tpu as pltpu
from jax.experimental.pallas import tpu_sc as plsc

_TOKENS = 2048
_D_IN = 768
_BASE = 128
_NL = _BASE * _BASE

_TC_ROWS = 1280
_CH = 128
_NBUF = 3
_NCH = _TC_ROWS // _CH

_SC_ROWS = _TOKENS - _TC_ROWS
_NW = 32
_RPW = _SC_ROWS // _NW

_HBM = pltpu.MemorySpace.HBM


def _alloc_body(o_ref):
    pass


def _alloc_out():
    return pl.pallas_call(
        _alloc_body,
        out_specs=pl.BlockSpec(memory_space=_HBM),
        out_shape=jax.ShapeDtypeStruct((_TOKENS, _NL), jnp.float32),
    )()


def _xs_kernel(x_ref, w_ref, b_ref, o_ref):
    o_ref[:] = jax.lax.dot_general(
        x_ref[:], w_ref[:],
        (((1,), (1,)), ((), ())),
        preferred_element_type=jnp.float32,
    ) + b_ref[:]


def _compute_xs_sc(x, W, b2):
    return pl.pallas_call(
        _xs_kernel,
        grid=(1,),
        in_specs=[
            pl.BlockSpec((_SC_ROWS, _D_IN), lambda i: (0, 0)),
            pl.BlockSpec((2 * _BASE, _D_IN), lambda i: (0, 0)),
            pl.BlockSpec((1, 2 * _BASE), lambda i: (0, 0)),
        ],
        out_specs=pl.BlockSpec((_SC_ROWS, 2 * _BASE), lambda i: (0, 0)),
        out_shape=jax.ShapeDtypeStruct((_SC_ROWS, 2 * _BASE), jnp.float32),
    )(x[_TC_ROWS:], W, b2)


def _tc_copy(buf, y_ref, sems, c):
    slot = c % _NBUF
    return pltpu.make_async_copy(
        buf.at[slot],
        y_ref.at[pl.ds(c * _CH, _CH), :],
        sems.at[slot],
    )


def _tc_body(x_ref, w_ref, b_ref, y_ref, d_ref, buf, sems):
    for c in range(_NCH):
        slot = c % _NBUF
        if c >= _NBUF:
            _tc_copy(buf, y_ref, sems, c - _NBUF).wait()
        xs = jax.lax.dot_general(
            x_ref[pl.ds(c * _CH, _CH), :], w_ref[:],
            (((1,), (1,)), ((), ())),
            preferred_element_type=jnp.float32,
        ) + b_ref[:]
        x1 = xs[:, :_BASE]
        x2 = xs[:, _BASE:]
        for i in range(_BASE):
            buf[slot, :, i * _BASE:(i + 1) * _BASE] = x1[:, i:i + 1] + x2
        _tc_copy(buf, y_ref, sems, c).start()
    for c in range(_NCH - _NBUF, _NCH):
        _tc_copy(buf, y_ref, sems, c).wait()
    d_ref[:] = jnp.zeros((8, 128), jnp.float32)


def _tc_write(x, W, b2, y0):
    return pl.pallas_call(
        _tc_body,
        grid=(1,),
        in_specs=[
            pl.BlockSpec((_TC_ROWS, _D_IN), lambda i: (0, 0)),
            pl.BlockSpec((2 * _BASE, _D_IN), lambda i: (0, 0)),
            pl.BlockSpec((1, 2 * _BASE), lambda i: (0, 0)),
            pl.BlockSpec(memory_space=_HBM),
        ],
        out_specs=pl.BlockSpec((8, 128), lambda i: (0, 0)),
        out_shape=jax.ShapeDtypeStruct((8, 128), jnp.float32),
        scratch_shapes=[
            pltpu.VMEM((_NBUF, _CH, _NL), jnp.float32),
            pltpu.SemaphoreType.DMA((_NBUF,)),
        ],
    )(x[:_TC_ROWS], W, b2, y0)


def _sc_body(xs_hbm, y_hbm, d_hbm, xs_v, buf, sem):
    wid = lax.axis_index("s") * 2 + lax.axis_index("c")
    base = wid * _RPW
    pltpu.sync_copy(xs_hbm.at[pl.ds(base, _RPW)], xs_v)

    def row_body(t, carry):
        slot = lax.rem(t, 2)

        @pl.when(t >= 2)
        def _wait_prev():
            pltpu.make_async_copy(
                buf.at[slot], y_hbm.at[_TC_ROWS + base + t - 2], sem
            ).wait()

        x2vs = [xs_v[t, pl.ds(_BASE + jv * 16, 16)] for jv in range(8)]

        def col_body(iv, c2):
            x1v = xs_v[t, pl.ds(iv * 16, 16)]
            for l in range(16):
                s = x1v[l]
                off = (iv * 16 + l) * _BASE
                for jv in range(8):
                    buf[slot, pl.ds(off + jv * 16, 16)] = s + x2vs[jv]
            return c2

        lax.fori_loop(0, 8, col_body, 0)
        pltpu.make_async_copy(
            buf.at[slot], y_hbm.at[_TC_ROWS + base + t], sem
        ).start()
        return carry

    lax.fori_loop(0, _RPW, row_body, 0)
    for t in (_RPW - 2, _RPW - 1):
        pltpu.make_async_copy(
            buf.at[t % 2], y_hbm.at[_TC_ROWS + base + t], sem
        ).wait()


def _sc_write(xs_sc, y0):
    mesh = plsc.VectorSubcoreMesh(core_axis_name="c", subcore_axis_name="s")
    return pl.kernel(
        _sc_body,
        out_type=jax.ShapeDtypeStruct((32, 16), jnp.float32),
        mesh=mesh,
        scratch_types=[
            pltpu.VMEM((_RPW, 2 * _BASE), jnp.float32),
            pltpu.VMEM((2, _NL), jnp.float32),
            pltpu.SemaphoreType.DMA,
        ],
    )(xs_sc, y0)


def _fin_body(y_ref, dsc_ref, dtc_ref, o_ref):
    pass


def _finalize(y0, d_sc, d_tc):
    return pl.pallas_call(
        _fin_body,
        grid=(1,),
        in_specs=[
            pl.BlockSpec(memory_space=_HBM),
            pl.BlockSpec((32, 16), lambda i: (0, 0)),
            pl.BlockSpec((8, 128), lambda i: (0, 0)),
        ],
        out_specs=pl.BlockSpec(memory_space=_HBM),
        out_shape=jax.ShapeDtypeStruct((_TOKENS, _NL), jnp.float32),
        input_output_aliases={0: 0},
    )(y0, d_sc, d_tc)


def kernel(x, W, b):
    b2 = b.reshape(1, 2 * _BASE)
    y0 = _alloc_out()
    xs_sc = _compute_xs_sc(x, W, b2)
    d_sc = _sc_write(xs_sc, y0)
    d_tc = _tc_write(x, W, b2, y0)
    return _finalize(y0, d_sc, d_tc)
